# baseline (speedup 1.0000x reference)
import jax
import jax.numpy as jnp
from jax.experimental import pallas as pl
from jax.experimental.pallas import tpu as pltpu

_N = 16384
_D = 512
_CHUNKS = 8
_ROWS_PER_CHUNK = _N // _CHUNKS


def _concat_body(idx_ref, x_hbm, w_hbm, out_hbm, copy_sem, row_sem):
    p = idx_ref[0]
    o = idx_ref[1]
    row_cp = pltpu.make_async_copy(
        w_hbm.at[p, o], out_hbm.at[pl.ds(_N, 1)], row_sem
    )
    row_cp.start()
    cps = []
    for c in range(_CHUNKS):
        sl = pl.ds(c * _ROWS_PER_CHUNK, _ROWS_PER_CHUNK)
        cp = pltpu.make_async_copy(x_hbm.at[sl], out_hbm.at[sl], copy_sem)
        cp.start()
        cps.append(cp)
    for cp in cps:
        cp.wait()
    row_cp.wait()


def kernel(input_features, part_cls, obj_cls, W):
    idx = jnp.stack(
        [jnp.asarray(part_cls, jnp.int32), jnp.asarray(obj_cls, jnp.int32)]
    )
    return pl.pallas_call(
        _concat_body,
        grid=(),
        in_specs=[
            pl.BlockSpec(memory_space=pltpu.SMEM),
            pl.BlockSpec(memory_space=pl.ANY),
            pl.BlockSpec(memory_space=pl.ANY),
        ],
        out_specs=pl.BlockSpec(memory_space=pl.ANY),
        out_shape=jax.ShapeDtypeStruct((_N + 1, _D), jnp.float32),
        scratch_shapes=[pltpu.SemaphoreType.DMA, pltpu.SemaphoreType.DMA],
    )(idx, input_features, W)


# speedup vs baseline: 40.9715x; 40.9715x over previous
import jax
import jax.numpy as jnp
from jax.experimental import pallas as pl
from jax.experimental.pallas import tpu as pltpu

_N = 16384
_D = 512
_BLK = 2048
_GRID = _N // _BLK + 1


def _concat_body(idx_ref, x_ref, w_ref, o_ref):
    i = pl.program_id(0)

    @pl.when(i < _GRID - 1)
    def _copy():
        o_ref[...] = x_ref[...]

    @pl.when(i == _GRID - 1)
    def _tail():
        o_ref[...] = jnp.broadcast_to(w_ref[0, 0], (_BLK, _D))


def kernel(input_features, part_cls, obj_cls, W):
    idx = jnp.stack(
        [jnp.asarray(part_cls, jnp.int32), jnp.asarray(obj_cls, jnp.int32)]
    )
    grid_spec = pltpu.PrefetchScalarGridSpec(
        num_scalar_prefetch=1,
        grid=(_GRID,),
        in_specs=[
            pl.BlockSpec(
                (_BLK, _D), lambda i, idx: (jnp.minimum(i, _N // _BLK - 1), 0)
            ),
            pl.BlockSpec((1, 1, 1, _D), lambda i, idx: (idx[0], idx[1], 0, 0)),
        ],
        out_specs=pl.BlockSpec((_BLK, _D), lambda i, idx: (i, 0)),
    )
    return pl.pallas_call(
        _concat_body,
        grid_spec=grid_spec,
        out_shape=jax.ShapeDtypeStruct((_N + 1, _D), jnp.float32),
    )(idx, input_features, W)


# speedup vs baseline: 43.6223x; 1.0647x over previous
import jax
import jax.numpy as jnp
from jax.experimental import pallas as pl
from jax.experimental.pallas import tpu as pltpu

_N = 16384
_D = 512
_BLK = 4096
_GRID = _N // _BLK + 1


def _concat_body(idx_ref, x_ref, w_ref, o_ref):
    i = pl.program_id(0)

    @pl.when(i < _GRID - 1)
    def _copy():
        o_ref[...] = x_ref[...]

    @pl.when(i == _GRID - 1)
    def _tail():
        o_ref[...] = jnp.broadcast_to(w_ref[0, 0], (_BLK, _D))


def kernel(input_features, part_cls, obj_cls, W):
    idx = jnp.stack(
        [jnp.asarray(part_cls, jnp.int32), jnp.asarray(obj_cls, jnp.int32)]
    )
    grid_spec = pltpu.PrefetchScalarGridSpec(
        num_scalar_prefetch=1,
        grid=(_GRID,),
        in_specs=[
            pl.BlockSpec(
                (_BLK, _D), lambda i, idx: (jnp.minimum(i, _N // _BLK - 1), 0)
            ),
            pl.BlockSpec((1, 1, 1, _D), lambda i, idx: (idx[0], idx[1], 0, 0)),
        ],
        out_specs=pl.BlockSpec((_BLK, _D), lambda i, idx: (i, 0)),
    )
    return pl.pallas_call(
        _concat_body,
        grid_spec=grid_spec,
        out_shape=jax.ShapeDtypeStruct((_N + 1, _D), jnp.float32),
    )(idx, input_features, W)
